# Initial kernel scaffold; baseline (speedup 1.0000x reference)
import jax
import jax.numpy as jnp
from jax.experimental import pallas as pl


def kernel(x, edge_index, edge_attr, batch, W0, as0, ad0, We0, ae0, b0, W1, as1, ad1, We1, ae1, b1, W2, as2, ad2, We2, ae2, b2, ln0_g, ln0_b, ln1_g, ln1_b, mp1_W, mp1_b, mp2_W, mp2_b):
    raise NotImplementedError("write your pallas kernel here")


# speedup vs baseline: 27.7961x; 27.7961x over previous
import functools

import jax
import jax.numpy as jnp
from jax import lax
from jax.experimental import pallas as pl
from jax.experimental.pallas import tpu as pltpu
from jax.experimental.pallas import tpu_sc as plsc

N = 10000
NP = 10240
D = 128
E = 320000
EPT = 10112
EP = EPT * 32
NBLKB = 158
KB = 64
G = 16
O = 64
F32 = jnp.float32


def _sca_body(src_hbm, dst_hbm, basec_hbm, ast_hbm, adt_hbm,
              p_hbm, den2_hbm,
              asrcT, adstT, den_local, srcb, dstb, bcb, pb):
    cid = lax.axis_index("c")
    sid = lax.axis_index("s")
    wid = sid * 2 + cid
    e0 = wid * EPT

    pltpu.sync_copy(ast_hbm, asrcT)
    pltpu.sync_copy(adt_hbm, adstT)
    pltpu.sync_copy(src_hbm.at[pl.ds(e0, EPT)], srcb)
    pltpu.sync_copy(dst_hbm.at[pl.ds(e0, EPT)], dstb)
    pltpu.sync_copy(basec_hbm.at[pl.ds(e0, EPT)], bcb)

    zero16 = jnp.zeros((16,), F32)

    def zden(i, c):
        den_local[pl.ds(i * 16, 16)] = zero16
        return c
    lax.fori_loop(0, NP // 16, zden, 0)

    def step(i, c):
        sl = pl.ds(i * 16, 16)
        s16 = srcb[sl]
        d16 = dstb[sl]
        b16 = bcb[sl]
        a1 = plsc.load_gather(asrcT, [s16])
        a2 = plsc.load_gather(adstT, [d16])
        t = a1 + a2 + b16
        t = jnp.where(t >= 0, t, t * F32(0.2))
        p16 = jnp.exp(t)
        pb[sl] = p16
        plsc.addupdate_scatter(den_local, [d16], p16)
        return c
    lax.fori_loop(0, EPT // 16, step, 0)

    pltpu.sync_copy(pb, p_hbm.at[pl.ds(e0, EPT)])
    pltpu.sync_copy(den_local, den2_hbm.at[wid])


_SC_EDGES_FN = None


def _sc_edges(*args):
    global _SC_EDGES_FN
    if _SC_EDGES_FN is None:
        mesh = plsc.VectorSubcoreMesh(core_axis_name="c",
                                      subcore_axis_name="s")
        _SC_EDGES_FN = functools.partial(
            pl.kernel,
            out_type=(jax.ShapeDtypeStruct((EP,), F32),
                      jax.ShapeDtypeStruct((32, NP), F32)),
            mesh=mesh,
            compiler_params=pltpu.CompilerParams(needs_layout_passes=False),
            scratch_types=(
                [pltpu.VMEM((NP,), F32),
                 pltpu.VMEM((NP,), F32),
                 pltpu.VMEM((NP,), F32),
                 pltpu.VMEM((EPT,), jnp.int32),
                 pltpu.VMEM((EPT,), jnp.int32),
                 pltpu.VMEM((EPT,), F32),
                 pltpu.VMEM((EPT,), F32)]
            ),
        )(_sca_body)
    return _SC_EDGES_FN(*args)


def _scb_body(src_hbm, dst_hbm, p_hbm, h_hbm,
              out2_hbm,
              rows0, rows1, rows2, rows3,
              si0, si1, si2, si3,
              di0, di1, di2, di3,
              pb0, pb1, pb2, pb3,
              acc,
              ig0, ig1, ig2, ig3,
              g0, g1, g2, g3,
              s0, s1, s2, s3):
    cid = lax.axis_index("c")
    sid = lax.axis_index("s")
    wid = sid * 2 + cid
    e0 = wid * EPT
    ROWS = (rows0, rows1, rows2, rows3)
    SI = (si0, si1, si2, si3)
    DI = (di0, di1, di2, di3)
    PB = (pb0, pb1, pb2, pb3)
    IG = (ig0, ig1, ig2, ig3)
    GS = (g0, g1, g2, g3)
    SS = (s0, s1, s2, s3)

    zero16 = jnp.zeros((16,), F32)

    def zrow(r, c):
        for cc in range(8):
            rows0[r, pl.ds(cc * 16, 16)] = zero16
        return c
    lax.fori_loop(0, KB, zrow, 0)

    for j in range(640 // KB):
        pltpu.sync_copy(rows0, acc.at[pl.ds(sid * 640 + j * KB, KB)])
    plsc.subcore_barrier()

    def idx_start(b, s):
        pltpu.async_copy(src_hbm.at[pl.ds(e0 + b * KB, KB)], SI[s], IG[s])
        pltpu.async_copy(dst_hbm.at[pl.ds(e0 + b * KB, KB)], DI[s], IG[s])
        pltpu.async_copy(p_hbm.at[pl.ds(e0 + b * KB, KB)], PB[s], IG[s])

    def idx_wait(b, s):
        pltpu.make_async_copy(
            src_hbm.at[pl.ds(e0 + b * KB, KB)], SI[s], IG[s]).wait()
        pltpu.make_async_copy(
            dst_hbm.at[pl.ds(e0 + b * KB, KB)], DI[s], IG[s]).wait()
        pltpu.make_async_copy(
            p_hbm.at[pl.ds(e0 + b * KB, KB)], PB[s], IG[s]).wait()

    def gather_start(s):
        pltpu.async_copy(h_hbm.at[SI[s]], ROWS[s], GS[s])

    def gather_wait(s):
        pltpu.make_async_copy(h_hbm.at[SI[s]], ROWS[s], GS[s]).wait()

    def scat_start(s):
        pltpu.async_copy(ROWS[s], acc.at[DI[s]], SS[s], add=True)

    def scat_wait(s):
        pltpu.make_async_copy(ROWS[s], acc.at[DI[s]], SS[s]).wait()

    def proc(b, s):
        gather_wait(s)

        def srow(jj, c):
            p16 = PB[s][pl.ds(jj * 16, 16)]
            for li in range(16):
                pvv = jnp.full((16,), p16[li], F32)
                r = jj * 16 + li
                for cc in range(8):
                    csl = pl.ds(cc * 16, 16)
                    ROWS[s][r, csl] = ROWS[s][r, csl] * pvv
            return c
        lax.fori_loop(0, KB // 16, srow, 0)
        scat_start(s)

    idx_start(0, 0)
    idx_start(1, 1)
    idx_wait(0, 0)
    gather_start(0)

    def super_body(g, c):
        for s in range(4):
            b = g * 4 + s

            @pl.when(b < NBLKB)
            def _():
                @pl.when(b >= 2)
                def _():
                    scat_wait((s + 2) % 4)

                @pl.when(b + 2 < NBLKB)
                def _():
                    idx_start(b + 2, (s + 2) % 4)

                @pl.when(b + 1 < NBLKB)
                def _():
                    idx_wait(b + 1, (s + 1) % 4)
                    gather_start((s + 1) % 4)

                proc(b, s)
        return c
    lax.fori_loop(0, (NBLKB + 3) // 4, super_body, 0)

    scat_wait((NBLKB - 2) % 4)
    scat_wait((NBLKB - 1) % 4)
    plsc.subcore_barrier()

    for j in range(640 // KB):
        sl = pl.ds(sid * 640 + j * KB, KB)
        pltpu.sync_copy(acc.at[sl], out2_hbm.at[cid, sl])


_SC_ROWS_FN = None


def _sc_rows(*args):
    global _SC_ROWS_FN
    if _SC_ROWS_FN is None:
        mesh = plsc.VectorSubcoreMesh(core_axis_name="c",
                                      subcore_axis_name="s")
        _SC_ROWS_FN = functools.partial(
            pl.kernel,
            out_type=jax.ShapeDtypeStruct((2, NP, D), F32),
            mesh=mesh,
            compiler_params=pltpu.CompilerParams(needs_layout_passes=False),
            scratch_types=(
                [pltpu.VMEM((KB, D), F32) for _ in range(4)]
                + [pltpu.VMEM((KB,), jnp.int32) for _ in range(8)]
                + [pltpu.VMEM((KB,), F32) for _ in range(4)]
                + [pltpu.VMEM_SHARED((NP, D), F32)]
                + [pltpu.SemaphoreType.DMA for _ in range(12)]
            ),
        )(_scb_body)
    return _SC_ROWS_FN(*args)


def _dense_pre_body(x_ref, w_ref, asr_ref, adr_ref, h_ref, ast_ref, adt_ref):
    h = jnp.dot(x_ref[...], w_ref[...], preferred_element_type=F32)
    h_ref[...] = h
    ast_ref[...] = lax.dot_general(
        asr_ref[...], h, (((1,), (1,)), ((), ()))).reshape(1, 1, 128)
    adt_ref[...] = lax.dot_general(
        adr_ref[...], h, (((1,), (1,)), ((), ()))).reshape(1, 1, 128)


def _dense_pre(xp, w, asr, adr):
    return pl.pallas_call(
        _dense_pre_body,
        grid=(NP // 128,),
        in_specs=[
            pl.BlockSpec((128, D), lambda i: (i, 0)),
            pl.BlockSpec((D, D), lambda i: (0, 0)),
            pl.BlockSpec((1, D), lambda i: (0, 0)),
            pl.BlockSpec((1, D), lambda i: (0, 0)),
        ],
        out_specs=[
            pl.BlockSpec((128, D), lambda i: (i, 0)),
            pl.BlockSpec((1, 1, 128), lambda i: (i, 0, 0)),
            pl.BlockSpec((1, 1, 128), lambda i: (i, 0, 0)),
        ],
        out_shape=[
            jax.ShapeDtypeStruct((NP, D), F32),
            jax.ShapeDtypeStruct((NP // 128, 1, 128), F32),
            jax.ShapeDtypeStruct((NP // 128, 1, 128), F32),
        ],
    )(xp, w, asr, adr)


def _edge_const_body(base_ref, we0, ae0, we1, ae1, we2, ae2,
                     b0_ref, b1_ref, b2_ref, mc_ref):
    base = base_ref[...]
    mean = jnp.sum(base) / F32(E)
    c0 = jnp.sum(we0[...] * ae0[...])
    c1 = jnp.sum(we1[...] * ae1[...])
    c2 = jnp.sum(we2[...] * ae2[...])
    b0_ref[...] = base * c0
    b1_ref[...] = base * c1
    b2_ref[...] = base * c2
    mc_ref[...] = jnp.concatenate(
        [jnp.full((1, 128), mean, F32),
         jnp.full((1, 128), c0, F32),
         jnp.full((1, 128), c1, F32),
         jnp.full((1, 128), c2, F32),
         jnp.zeros((4, 128), F32)], axis=0)


def _edge_const(base2d, we0, ae0, we1, ae1, we2, ae2):
    return pl.pallas_call(
        _edge_const_body,
        out_shape=[
            jax.ShapeDtypeStruct((EP // 128, 128), F32),
            jax.ShapeDtypeStruct((EP // 128, 128), F32),
            jax.ShapeDtypeStruct((EP // 128, 128), F32),
            jax.ShapeDtypeStruct((8, 128), F32),
        ],
    )(base2d, we0, ae0, we1, ae1, we2, ae2)


def _gat_combine(out2, den2, h, asw, adw, mc, lidx):
    acc = out2[0] + out2[1]
    ones32 = jnp.ones((32, 1), F32)
    den = lax.dot_general(den2, ones32, (((0,), (0,)), ((), ())))
    ascol = jnp.dot(h, asw, preferred_element_type=F32)
    adcol = jnp.dot(h, adw, preferred_element_type=F32)
    cm = mc[lidx + 1:lidx + 2, 0:1] * mc[0:1, 0:1]
    t = ascol + adcol + cm
    t = jnp.where(t >= 0, t, t * F32(0.2))
    ps = jnp.exp(t)
    num = acc + ps * h
    return num / (den + ps + F32(1e-16))


def _dense_mid_body(lidx, out2_ref, den2_ref, h_ref, asw_ref, adw_ref,
                    mc_ref, b_ref, g_ref, lb_ref, wn_ref, asn_ref, adn_ref,
                    hn_ref, astn_ref, adtn_ref):
    o = _gat_combine(out2_ref[...], den2_ref[...], h_ref[...],
                     asw_ref[...], adw_ref[...], mc_ref[...], lidx)
    o = jnp.maximum(o + b_ref[...], 0.0)
    mu = jnp.mean(o, axis=1, keepdims=True)
    v = jnp.mean((o - mu) * (o - mu), axis=1, keepdims=True)
    o = (o - mu) * lax.rsqrt(v + F32(1e-5)) * g_ref[...] + lb_ref[...]
    hn = jnp.dot(o, wn_ref[...], preferred_element_type=F32)
    hn_ref[...] = hn
    astn_ref[...] = lax.dot_general(
        asn_ref[...], hn, (((1,), (1,)), ((), ()))).reshape(1, 1, 128)
    adtn_ref[...] = lax.dot_general(
        adn_ref[...], hn, (((1,), (1,)), ((), ()))).reshape(1, 1, 128)


def _dense_mid(lidx, out2, den2, h, asw, adw, mc, b, g, lb, wn, asn, adn):
    return pl.pallas_call(
        functools.partial(_dense_mid_body, lidx),
        grid=(NP // 128,),
        in_specs=[
            pl.BlockSpec((2, 128, D), lambda i: (0, i, 0)),
            pl.BlockSpec((32, 128), lambda i: (0, i)),
            pl.BlockSpec((128, D), lambda i: (i, 0)),
            pl.BlockSpec((D, 1), lambda i: (0, 0)),
            pl.BlockSpec((D, 1), lambda i: (0, 0)),
            pl.BlockSpec((8, 128), lambda i: (0, 0)),
            pl.BlockSpec((1, D), lambda i: (0, 0)),
            pl.BlockSpec((1, D), lambda i: (0, 0)),
            pl.BlockSpec((1, D), lambda i: (0, 0)),
            pl.BlockSpec((D, D), lambda i: (0, 0)),
            pl.BlockSpec((1, D), lambda i: (0, 0)),
            pl.BlockSpec((1, D), lambda i: (0, 0)),
        ],
        out_specs=[
            pl.BlockSpec((128, D), lambda i: (i, 0)),
            pl.BlockSpec((1, 1, 128), lambda i: (i, 0, 0)),
            pl.BlockSpec((1, 1, 128), lambda i: (i, 0, 0)),
        ],
        out_shape=[
            jax.ShapeDtypeStruct((NP, D), F32),
            jax.ShapeDtypeStruct((NP // 128, 1, 128), F32),
            jax.ShapeDtypeStruct((NP // 128, 1, 128), F32),
        ],
    )(out2, den2, h, asw, adw, mc, b, g, lb, wn, asn, adn)


def _dense_final_body(out2_ref, den2_ref, h_ref, asw_ref, adw_ref, mc_ref,
                      b_ref, batch_ref, w1_ref, b1_ref, w2_ref, b2_ref,
                      out_ref):
    o = _gat_combine(out2_ref[...], den2_ref[...], h_ref[...],
                     asw_ref[...], adw_ref[...], mc_ref[...], 2)
    o = jnp.maximum(o + b_ref[...], 0.0)
    grp = batch_ref[...]
    gi = lax.broadcasted_iota(jnp.int32, (1, G), 1)
    onehot = (grp == gi).astype(F32)
    cnt = lax.dot_general(onehot, jnp.ones((NP, 1), F32),
                          (((0,), (0,)), ((), ())))
    pooled = lax.dot_general(onehot, o, (((0,), (0,)), ((), ())))
    pooled = pooled / jnp.maximum(cnt, 1.0)
    z = jnp.dot(pooled, w1_ref[...], preferred_element_type=F32) + b1_ref[...]
    z = jnp.dot(z, w2_ref[...], preferred_element_type=F32) + b2_ref[...]
    zm = jnp.max(z, axis=1, keepdims=True)
    zs = z - zm
    out_ref[...] = zs - jnp.log(jnp.sum(jnp.exp(zs), axis=1, keepdims=True))


def _dense_final(out2, den2, h, asw, adw, mc, b, batch2d, w1, b1, w2, b2):
    return pl.pallas_call(
        _dense_final_body,
        out_shape=jax.ShapeDtypeStruct((G, O), F32),
    )(out2, den2, h, asw, adw, mc, b, batch2d, w1, b1, w2, b2)


def kernel(x, edge_index, edge_attr, batch,
           W0, as0, ad0, We0, ae0, b0,
           W1, as1, ad1, We1, ae1, b1,
           W2, as2, ad2, We2, ae2, b2,
           ln0_g, ln0_b, ln1_g, ln1_b,
           mp1_W, mp1_b, mp2_W, mp2_b):
    i32 = jnp.int32
    xp = jnp.pad(x, ((0, NP - N), (0, 0)))
    srcp = jnp.concatenate(
        [edge_index[0], jnp.full((EP - E,), NP - 1, i32)])
    dstp = jnp.concatenate(
        [edge_index[1], jnp.full((EP - E,), NP - 1, i32)])
    basep = jnp.concatenate([edge_attr[:, 0], jnp.zeros((EP - E,), F32)])
    base2d = basep.reshape(EP // 128, 128)
    batch2d = jnp.pad(batch, (0, NP - N), constant_values=G).reshape(NP, 1)

    r1 = lambda a: a.reshape(1, D)
    c1 = lambda a: a.reshape(D, 1)
    rT = lambda a: a.reshape(NP)

    basec0, basec1, basec2, mc = _edge_const(
        base2d, We0, r1(ae0), We1, r1(ae1), We2, r1(ae2))

    h0, ast0, adt0 = _dense_pre(xp, W0, r1(as0), r1(ad0))
    pv, den2 = _sc_edges(srcp, dstp, basec0.reshape(EP), rT(ast0), rT(adt0))
    out2 = _sc_rows(srcp, dstp, pv, h0)
    h1, ast1, adt1 = _dense_mid(0, out2, den2, h0, c1(as0), c1(ad0), mc,
                                r1(b0), r1(ln0_g), r1(ln0_b), W1,
                                r1(as1), r1(ad1))
    pv, den2 = _sc_edges(srcp, dstp, basec1.reshape(EP), rT(ast1), rT(adt1))
    out2 = _sc_rows(srcp, dstp, pv, h1)
    h2, ast2, adt2 = _dense_mid(1, out2, den2, h1, c1(as1), c1(ad1), mc,
                                r1(b1), r1(ln1_g), r1(ln1_b), W2,
                                r1(as2), r1(ad2))
    pv, den2 = _sc_edges(srcp, dstp, basec2.reshape(EP), rT(ast2), rT(adt2))
    out2 = _sc_rows(srcp, dstp, pv, h2)
    return _dense_final(out2, den2, h2, c1(as2), c1(ad2), mc, r1(b2),
                        batch2d, mp1_W, r1(mp1_b), mp2_W,
                        mp2_b.reshape(1, O))
